# baseline (speedup 1.0000x reference)
import functools

import jax
import jax.numpy as jnp
from jax import lax
from jax.experimental import pallas as pl
from jax.experimental.pallas import tpu as pltpu
from jax.experimental.pallas import tpu_sc as plsc

VOCAB = 400000
EMB_DIM = 100
HIDDEN = 128
NUM_CLASSES = 4
BATCH = 4096
SEQLEN = 200

DP = 128
NC = 2
NS = 16
NW = NC * NS
CHUNK = 100
CP = 104
CPW = (BATCH * SEQLEN) // (NW * CHUNK)
NBUF = 4
RPW = BATCH // NW
LANES = 16
NBLK = DP // 32
OUT_OFFS = (0, 16, 32, 48, 64, 80, 84)


def _pool_body(x_hbm, tab_hbm, pooled_hbm, idx_v, b0, b1, b2, b3, acc,
               out_v, s0, s1, s2, s3):
    cid = lax.axis_index("c")
    sid = lax.axis_index("s")
    wid = sid * NC + cid
    cbase = wid * CPW

    bufs = (b0, b1, b2, b3)
    sems = (s0, s1, s2, s3)

    pltpu.sync_copy(x_hbm.at[pl.ds(cbase, CPW)], idx_v)

    zvec = jnp.zeros((LANES,), jnp.float32)
    himask = jnp.full((LANES,), -65536, jnp.int32)

    def fire(c, k):
        pltpu.async_copy(tab_hbm.at[idx_v.at[c]], bufs[k], sems[k])

    def wait_all(c, k):
        pltpu.make_async_copy(tab_hbm.at[idx_v.at[c]], bufs[k],
                              sems[k]).wait()

    def accumulate(buf, carry_in):
        @pl.loop(0, CHUNK // 2, init_carry=carry_in)
        def carry_out(g, carry):
            vs = list(carry)
            for rr in range(2):
                r = g * 2 + rr
                for b in range(NBLK):
                    w = plsc.bitcast(buf[r, pl.ds(b * 32, 32)], jnp.int32)
                    lo = plsc.bitcast(w << 16, jnp.float32)
                    hi = plsc.bitcast(w & himask, jnp.float32)
                    vs[2 * b] = vs[2 * b] + lo
                    vs[2 * b + 1] = vs[2 * b + 1] + hi
            return tuple(vs)

        return carry_out

    for k in range(NBUF):
        fire(k, k)

    def do_chunk(c, k, carry):
        wait_all(c, k)
        carry = accumulate(bufs[k], carry)
        fire(jnp.minimum(c + NBUF, CPW - NBUF + k), k)
        return carry

    evens = lax.iota(jnp.int32, LANES) * 2
    odds = evens + 1

    def store_row(i, acc8):
        for b in range(NBLK):
            base = 32 * b
            plsc.store_scatter(acc, [evens + base], acc8[2 * b])
            plsc.store_scatter(acc, [odds + base], acc8[2 * b + 1])
        for off in OUT_OFFS:
            out_v[i, pl.ds(off, LANES)] = acc[pl.ds(off, LANES)]

    zero8 = (zvec,) * (2 * NBLK)

    @pl.loop(0, RPW // 2)
    def _(q):
        c0 = 4 * q
        acc8 = do_chunk(c0, 0, zero8)
        acc8 = do_chunk(c0 + 1, 1, acc8)
        store_row(2 * q, acc8)

        acc8 = do_chunk(c0 + 2, 2, zero8)
        acc8 = do_chunk(c0 + 3, 3, acc8)
        store_row(2 * q + 1, acc8)

    for k in range(NBUF):
        wait_all(CPW - NBUF + k, k)

    pltpu.sync_copy(out_v, pooled_hbm.at[pl.ds(wid * RPW, RPW)])


@functools.partial(
    pl.kernel,
    out_type=jax.ShapeDtypeStruct((BATCH, EMB_DIM), jnp.float32),
    mesh=plsc.VectorSubcoreMesh(core_axis_name="c", subcore_axis_name="s"),
    compiler_params=pltpu.CompilerParams(use_tc_tiling_on_sc=False,
                                         needs_layout_passes=False),
    scratch_types=[
        pltpu.VMEM((CPW, CP), jnp.int32),
        pltpu.VMEM((CP, DP), jnp.bfloat16),
        pltpu.VMEM((CP, DP), jnp.bfloat16),
        pltpu.VMEM((CP, DP), jnp.bfloat16),
        pltpu.VMEM((CP, DP), jnp.bfloat16),
        pltpu.VMEM((DP,), jnp.float32),
        pltpu.VMEM((RPW, EMB_DIM), jnp.float32),
        pltpu.SemaphoreType.DMA,
        pltpu.SemaphoreType.DMA,
        pltpu.SemaphoreType.DMA,
        pltpu.SemaphoreType.DMA,
    ],
)
def _pool(x_hbm, tab_hbm, pooled_hbm, *rest):
    _pool_body(x_hbm, tab_hbm, pooled_hbm, *rest)


def _mlp_body(p_ref, w1_ref, b1_ref, w2_ref, b2_ref, o_ref):
    h = jnp.dot(p_ref[...], w1_ref[...], preferred_element_type=jnp.float32)
    h = h * (1.0 / SEQLEN) + b1_ref[...]
    h = jnp.maximum(h, 0.0)
    o_ref[...] = (
        jnp.dot(h, w2_ref[...], preferred_element_type=jnp.float32)
        + b2_ref[...]
    )


_mlp = pl.pallas_call(
    _mlp_body,
    out_shape=jax.ShapeDtypeStruct((BATCH, NUM_CLASSES), jnp.float32),
)

_PAD_ROWS = 4000


def _padcast_body(t_ref, o_ref):
    o_ref[:, :EMB_DIM] = t_ref[...].astype(jnp.bfloat16)
    o_ref[:, EMB_DIM:] = jnp.zeros((_PAD_ROWS, DP - EMB_DIM), jnp.bfloat16)


_padcast = pl.pallas_call(
    _padcast_body,
    grid=(VOCAB // _PAD_ROWS,),
    in_specs=[pl.BlockSpec((_PAD_ROWS, EMB_DIM), lambda i: (i, 0))],
    out_specs=pl.BlockSpec((_PAD_ROWS, DP), lambda i: (i, 0)),
    out_shape=jax.ShapeDtypeStruct((VOCAB, DP), jnp.bfloat16),
)


@jax.jit
def kernel(x, emb_table, W1, b1, W2, b2):
    tabp = _padcast(emb_table)
    xp = jnp.pad(x.reshape(-1, CHUNK), ((0, 0), (0, CP - CHUNK)))
    pooled = _pool(xp, tabp)
    return _mlp(pooled, W1, b1.reshape(1, HIDDEN), W2,
                b2.reshape(1, NUM_CLASSES))


# speedup vs baseline: 1.0409x; 1.0409x over previous
import functools

import jax
import jax.numpy as jnp
from jax import lax
from jax.experimental import pallas as pl
from jax.experimental.pallas import tpu as pltpu
from jax.experimental.pallas import tpu_sc as plsc

VOCAB = 400000
EMB_DIM = 100
HIDDEN = 128
NUM_CLASSES = 4
BATCH = 4096
SEQLEN = 200

DP = 128
NC = 2
NS = 16
NW = NC * NS
CHUNK = 100
CP = 104
CPW = (BATCH * SEQLEN) // (NW * CHUNK)
NBUF = 4
RPW = BATCH // NW
LANES = 16
NBLK = DP // 32
OUT_OFFS = (0, 16, 32, 48, 64, 80, 84)


def _pool_body(x_hbm, tab_hbm, pooled_hbm, idx_v, b0, b1, b2, b3, acc,
               out_v, s0, s1, s2, s3):
    cid = lax.axis_index("c")
    sid = lax.axis_index("s")
    wid = sid * NC + cid
    cbase = wid * CPW

    bufs = (b0, b1, b2, b3)
    sems = (s0, s1, s2, s3)

    pltpu.sync_copy(x_hbm.at[pl.ds(cbase, CPW)], idx_v)

    zvec = jnp.zeros((LANES,), jnp.float32)
    himask = jnp.full((LANES,), -65536, jnp.int32)

    def fire(c, k):
        pltpu.async_copy(tab_hbm.at[idx_v.at[c]], bufs[k], sems[k])

    def wait_all(c, k):
        pltpu.make_async_copy(tab_hbm.at[idx_v.at[c]], bufs[k],
                              sems[k]).wait()

    def accumulate(buf, carry_in):
        @pl.loop(0, CHUNK // 2, init_carry=carry_in)
        def carry_out(g, carry):
            vs = list(carry)
            for rr in range(2):
                r = g * 2 + rr
                for b in range(NBLK):
                    w = plsc.bitcast(buf[r, pl.ds(b * 32, 32)], jnp.int32)
                    lo = plsc.bitcast(w << 16, jnp.float32)
                    hi = plsc.bitcast(w & himask, jnp.float32)
                    vs[2 * b] = vs[2 * b] + lo
                    vs[2 * b + 1] = vs[2 * b + 1] + hi
            return tuple(vs)

        return carry_out

    for k in range(NBUF):
        fire(k, k)

    def do_chunk(c, k, carry):
        wait_all(c, k)
        carry = accumulate(bufs[k], carry)
        fire(jnp.minimum(c + NBUF, CPW - NBUF + k), k)
        return carry

    evens = lax.iota(jnp.int32, LANES) * 2
    odds = evens + 1

    def store_row(i, acc8):
        for b in range(NBLK):
            base = 32 * b
            plsc.store_scatter(acc, [evens + base], acc8[2 * b])
            plsc.store_scatter(acc, [odds + base], acc8[2 * b + 1])
        for off in OUT_OFFS:
            out_v[i, pl.ds(off, LANES)] = acc[pl.ds(off, LANES)]

    zero8 = (zvec,) * (2 * NBLK)

    @pl.loop(0, RPW // 2)
    def _(q):
        c0 = 4 * q
        acc8 = do_chunk(c0, 0, zero8)
        acc8 = do_chunk(c0 + 1, 1, acc8)
        store_row(2 * q, acc8)

        acc8 = do_chunk(c0 + 2, 2, zero8)
        acc8 = do_chunk(c0 + 3, 3, acc8)
        store_row(2 * q + 1, acc8)

    for k in range(NBUF):
        wait_all(CPW - NBUF + k, k)

    pltpu.sync_copy(out_v, pooled_hbm.at[pl.ds(wid * RPW, RPW)])


@functools.partial(
    pl.kernel,
    out_type=jax.ShapeDtypeStruct((BATCH, EMB_DIM), jnp.float32),
    mesh=plsc.VectorSubcoreMesh(core_axis_name="c", subcore_axis_name="s"),
    compiler_params=pltpu.CompilerParams(use_tc_tiling_on_sc=False,
                                         needs_layout_passes=False),
    scratch_types=[
        pltpu.VMEM((CPW, CP), jnp.int32),
        pltpu.VMEM((CP, DP), jnp.bfloat16),
        pltpu.VMEM((CP, DP), jnp.bfloat16),
        pltpu.VMEM((CP, DP), jnp.bfloat16),
        pltpu.VMEM((CP, DP), jnp.bfloat16),
        pltpu.VMEM((DP,), jnp.float32),
        pltpu.VMEM((RPW, EMB_DIM), jnp.float32),
        pltpu.SemaphoreType.DMA,
        pltpu.SemaphoreType.DMA,
        pltpu.SemaphoreType.DMA,
        pltpu.SemaphoreType.DMA,
    ],
)
def _pool(x_hbm, tab_hbm, pooled_hbm, *rest):
    _pool_body(x_hbm, tab_hbm, pooled_hbm, *rest)


def _mlp_body(p_ref, w1_ref, b1_ref, w2_ref, b2_ref, o_ref):
    h = jnp.dot(p_ref[...], w1_ref[...], preferred_element_type=jnp.float32)
    h = h * (1.0 / SEQLEN) + b1_ref[...]
    h = jnp.maximum(h, 0.0)
    o_ref[...] = (
        jnp.dot(h, w2_ref[...], preferred_element_type=jnp.float32)
        + b2_ref[...]
    )


_mlp = pl.pallas_call(
    _mlp_body,
    out_shape=jax.ShapeDtypeStruct((BATCH, NUM_CLASSES), jnp.float32),
)

@jax.jit
def kernel(x, emb_table, W1, b1, W2, b2):
    tabp = jnp.pad(emb_table, ((0, 0), (0, DP - EMB_DIM))).astype(jnp.bfloat16)
    xp = jnp.pad(x.reshape(-1, CHUNK), ((0, 0), (0, CP - CHUNK)))
    pooled = _pool(xp, tabp)
    return _mlp(pooled, W1, b1.reshape(1, HIDDEN), W2,
                b2.reshape(1, NUM_CLASSES))


# speedup vs baseline: 1.3148x; 1.2631x over previous
import functools

import jax
import jax.numpy as jnp
from jax import lax
from jax.experimental import pallas as pl
from jax.experimental.pallas import tpu as pltpu
from jax.experimental.pallas import tpu_sc as plsc

VOCAB = 400000
EMB_DIM = 100
HIDDEN = 128
NUM_CLASSES = 4
BATCH = 4096
SEQLEN = 200

DP = 128
NC = 2
NS = 16
NW = NC * NS
CP = 128
CPW = (BATCH * SEQLEN) // (NW * CP)
RPW = BATCH // NW
LANES = 16
NBLK = DP // 32
OUT_OFFS = (0, 16, 32, 48, 64, 80, 84)


def _pool_body(x_hbm, tab_hbm, pooled_hbm, idx_v, buf0, buf1, acc, out_v,
               sem0, sem1):
    cid = lax.axis_index("c")
    sid = lax.axis_index("s")
    wid = sid * NC + cid
    cbase = wid * CPW

    bufs = (buf0, buf1)
    sems = (sem0, sem1)

    pltpu.sync_copy(x_hbm.at[pl.ds(cbase, CPW)], idx_v)

    zvec = jnp.zeros((LANES,), jnp.float32)
    himask = jnp.full((LANES,), -65536, jnp.int32)

    def fire(c, k):
        pltpu.async_copy(tab_hbm.at[idx_v.at[c]], bufs[k], sems[k])

    def wait_all(c, k):
        pltpu.make_async_copy(tab_hbm.at[idx_v.at[c]], bufs[k],
                              sems[k]).wait()

    evens = lax.iota(jnp.int32, LANES) * 2
    odds = evens + 1

    def store_row(i, acc8):
        for b in range(NBLK):
            base = 32 * b
            plsc.store_scatter(acc, [evens + base], acc8[2 * b])
            plsc.store_scatter(acc, [odds + base], acc8[2 * b + 1])
        for off in OUT_OFFS:
            out_v[i, pl.ds(off, LANES)] = acc[pl.ds(off, LANES)]

    def accumulate(buf, c, carry_in):
        p = lax.rem(200 - lax.rem(128 * c, 200), 200)

        @pl.loop(0, CP // 2, init_carry=carry_in)
        def carry_out(g, carry):
            vs = list(carry[:-1])
            rc = carry[-1]
            is_b = (2 * g) == p

            @pl.when(jnp.logical_and(is_b, rc >= 0))
            def _():
                store_row(rc, vs)

            vs = [jnp.where(is_b, 0.0, v) for v in vs]
            rc = rc + is_b.astype(jnp.int32)
            for rr in range(2):
                r = g * 2 + rr
                for b in range(NBLK):
                    w = plsc.bitcast(buf[r, pl.ds(b * 32, 32)], jnp.int32)
                    lo = plsc.bitcast(w << 16, jnp.float32)
                    hi = plsc.bitcast(w & himask, jnp.float32)
                    vs[2 * b] = vs[2 * b] + lo
                    vs[2 * b + 1] = vs[2 * b + 1] + hi
            return (*vs, rc)

        return carry_out

    fire(0, 0)
    fire(1, 1)

    def do_chunk(c, k, carry):
        wait_all(c, k)
        carry = accumulate(bufs[k], c, carry)
        fire(jnp.minimum(c + 2, CPW - 2 + k), k)
        return carry

    zero8 = (zvec,) * (2 * NBLK)
    init = (*zero8, jnp.int32(-1))

    @pl.loop(0, CPW // 2, init_carry=init)
    def final(q, carry):
        carry = do_chunk(2 * q, 0, carry)
        carry = do_chunk(2 * q + 1, 1, carry)
        return carry

    store_row(RPW - 1, list(final[:-1]))

    wait_all(CPW - 2, 0)
    wait_all(CPW - 1, 1)

    pltpu.sync_copy(out_v, pooled_hbm.at[pl.ds(wid * RPW, RPW)])


@functools.partial(
    pl.kernel,
    out_type=jax.ShapeDtypeStruct((BATCH, EMB_DIM), jnp.float32),
    mesh=plsc.VectorSubcoreMesh(core_axis_name="c", subcore_axis_name="s"),
    compiler_params=pltpu.CompilerParams(use_tc_tiling_on_sc=False,
                                         needs_layout_passes=False),
    scratch_types=[
        pltpu.VMEM((CPW, CP), jnp.int32),
        pltpu.VMEM((CP, DP), jnp.bfloat16),
        pltpu.VMEM((CP, DP), jnp.bfloat16),
        pltpu.VMEM((DP,), jnp.float32),
        pltpu.VMEM((RPW, EMB_DIM), jnp.float32),
        pltpu.SemaphoreType.DMA,
        pltpu.SemaphoreType.DMA,
    ],
)
def _pool(x_hbm, tab_hbm, pooled_hbm, *rest):
    _pool_body(x_hbm, tab_hbm, pooled_hbm, *rest)


def _mlp_body(p_ref, w1_ref, b1_ref, w2_ref, b2_ref, o_ref):
    h = jnp.dot(p_ref[...], w1_ref[...], preferred_element_type=jnp.float32)
    h = h * (1.0 / SEQLEN) + b1_ref[...]
    h = jnp.maximum(h, 0.0)
    o_ref[...] = (
        jnp.dot(h, w2_ref[...], preferred_element_type=jnp.float32)
        + b2_ref[...]
    )


_mlp = pl.pallas_call(
    _mlp_body,
    out_shape=jax.ShapeDtypeStruct((BATCH, NUM_CLASSES), jnp.float32),
)

@jax.jit
def kernel(x, emb_table, W1, b1, W2, b2):
    tabp = jnp.pad(emb_table, ((0, 0), (0, DP - EMB_DIM))).astype(jnp.bfloat16)
    xp = x.reshape(-1, CP)
    pooled = _pool(xp, tabp)
    return _mlp(pooled, W1, b1.reshape(1, HIDDEN), W2,
                b2.reshape(1, NUM_CLASSES))
